# baseline (speedup 1.0000x reference)
import jax
import jax.numpy as jnp
from jax import lax
from jax.experimental import pallas as pl
from jax.experimental.pallas import tpu as pltpu
from jax.experimental.pallas import tpu_sc as plsc

NC = 2
NS = 16
L = 16
NW = NC * NS
B = 16384
D = 32
BPW = B // NW
CHUNK = 128
NCH = BPW // CHUNK
GROUPS = BPW // L
BW = 16


def _sc_kernel(users_hbm, pos_hbm, neg_hbm, uhi_hbm, phi_hbm, nhi_hbm,
               ue_hbm, ie_hbm, ub_hbm, ib_hbm,
               outp_hbm, outn_hbm,
               uidx, pidx, nidx, uhi, phi, nhi,
               ue_rows, pe_rows, ne_rows,
               ub_rows, pb_rows, nb_rows, outp_v, outn_v, sem):
    wid = lax.axis_index("core") * NS + lax.axis_index("subcore")

    pltpu.sync_copy(users_hbm.at[wid], uidx)
    pltpu.sync_copy(pos_hbm.at[wid], pidx)
    pltpu.sync_copy(neg_hbm.at[wid], nidx)
    pltpu.sync_copy(uhi_hbm.at[wid], uhi)
    pltpu.sync_copy(phi_hbm.at[wid], phi)
    pltpu.sync_copy(nhi_hbm.at[wid], nhi)

    copies = []
    for j in range(NCH):
        sl = pl.ds(j * CHUNK, CHUNK)
        copies.append(pltpu.async_copy(ue_hbm.at[uidx.at[j]], ue_rows.at[sl], sem))
        copies.append(pltpu.async_copy(ie_hbm.at[pidx.at[j]], pe_rows.at[sl], sem))
        copies.append(pltpu.async_copy(ie_hbm.at[nidx.at[j]], ne_rows.at[sl], sem))
        copies.append(pltpu.async_copy(ub_hbm.at[uhi.at[j]], ub_rows.at[sl], sem))
        copies.append(pltpu.async_copy(ib_hbm.at[phi.at[j]], pb_rows.at[sl], sem))
        copies.append(pltpu.async_copy(ib_hbm.at[nhi.at[j]], nb_rows.at[sl], sem))
    for c in copies:
        c.wait()

    lanes = lax.iota(jnp.int32, L)

    @pl.loop(0, GROUPS)
    def _group(g):
        pos = g * L + lanes
        irow = lax.shift_right_logical(pos, 7)
        icol = lax.bitwise_and(pos, 127)
        accp = jnp.zeros((L,), jnp.float32)
        accn = jnp.zeros((L,), jnp.float32)
        for dd in range(D):
            col = jnp.full((L,), dd, jnp.int32)
            u = plsc.load_gather(ue_rows, [pos, col])
            p = plsc.load_gather(pe_rows, [pos, col])
            n = plsc.load_gather(ne_rows, [pos, col])
            accp = accp + u * p
            accn = accn + u * n
        iu = plsc.load_gather(uidx, [irow, icol])
        ip = plsc.load_gather(pidx, [irow, icol])
        inn = plsc.load_gather(nidx, [irow, icol])
        ub = plsc.load_gather(ub_rows, [pos, lax.bitwise_and(iu, BW - 1)])
        pb = plsc.load_gather(pb_rows, [pos, lax.bitwise_and(ip, BW - 1)])
        nb = plsc.load_gather(nb_rows, [pos, lax.bitwise_and(inn, BW - 1)])
        outp_v[pl.ds(g * L, L)] = accp + ub + pb
        outn_v[pl.ds(g * L, L)] = accn + ub + nb

    pltpu.sync_copy(outp_v, outp_hbm.at[pl.ds(wid * BPW, BPW)])
    pltpu.sync_copy(outn_v, outn_hbm.at[pl.ds(wid * BPW, BPW)])


def kernel(users, pos_items, neg_items, user_embeddings, item_embeddings,
           user_biases, item_biases):
    users = users.astype(jnp.int32)
    pos_items = pos_items.astype(jnp.int32)
    neg_items = neg_items.astype(jnp.int32)
    users3 = users.reshape(NW, NCH, CHUNK)
    pos3 = pos_items.reshape(NW, NCH, CHUNK)
    neg3 = neg_items.reshape(NW, NCH, CHUNK)
    uhi3 = (users >> 4).reshape(NW, NCH, CHUNK)
    phi3 = (pos_items >> 4).reshape(NW, NCH, CHUNK)
    nhi3 = (neg_items >> 4).reshape(NW, NCH, CHUNK)
    ub16 = user_biases.reshape(-1, BW)
    ib16 = item_biases.reshape(-1, BW)

    mesh = plsc.VectorSubcoreMesh(core_axis_name="core",
                                  subcore_axis_name="subcore",
                                  num_cores=NC, num_subcores=NS)
    f = pl.kernel(
        _sc_kernel,
        compiler_params=pltpu.CompilerParams(needs_layout_passes=False,
                                             use_tc_tiling_on_sc=False),
        out_type=(jax.ShapeDtypeStruct((B,), jnp.float32),
                  jax.ShapeDtypeStruct((B,), jnp.float32)),
        mesh=mesh,
        scratch_types=[
            pltpu.VMEM((NCH, CHUNK), jnp.int32),
            pltpu.VMEM((NCH, CHUNK), jnp.int32),
            pltpu.VMEM((NCH, CHUNK), jnp.int32),
            pltpu.VMEM((NCH, CHUNK), jnp.int32),
            pltpu.VMEM((NCH, CHUNK), jnp.int32),
            pltpu.VMEM((NCH, CHUNK), jnp.int32),
            pltpu.VMEM((BPW, D), jnp.float32),
            pltpu.VMEM((BPW, D), jnp.float32),
            pltpu.VMEM((BPW, D), jnp.float32),
            pltpu.VMEM((BPW, BW), jnp.float32),
            pltpu.VMEM((BPW, BW), jnp.float32),
            pltpu.VMEM((BPW, BW), jnp.float32),
            pltpu.VMEM((BPW,), jnp.float32),
            pltpu.VMEM((BPW,), jnp.float32),
            pltpu.SemaphoreType.DMA,
        ],
    )
    pos_preds, neg_preds = f(users3, pos3, neg3, uhi3, phi3, nhi3,
                             user_embeddings, item_embeddings, ub16, ib16)
    return pos_preds, neg_preds


# speedup vs baseline: 1.0021x; 1.0021x over previous
import jax
import jax.numpy as jnp
from jax import lax
from jax.experimental import pallas as pl
from jax.experimental.pallas import tpu as pltpu
from jax.experimental.pallas import tpu_sc as plsc

NC = 2
NS = 16
L = 16
NW = NC * NS
B = 16384
D = 32
BPW = B // NW
CHUNK = 128
NCH = BPW // CHUNK
GROUPS = BPW // L


def _sc_kernel(users_hbm, pos_hbm, neg_hbm, ue_hbm, ie_hbm, ub_hbm, ib_hbm,
               outp_hbm, outn_hbm,
               uidx, pidx, nidx, ue_rows, pe_rows, ne_rows,
               ub_v, pb_v, nb_v, outp_v, outn_v, sem):
    wid = lax.axis_index("core") * NS + lax.axis_index("subcore")

    pltpu.sync_copy(users_hbm.at[wid], uidx)
    pltpu.sync_copy(pos_hbm.at[wid], pidx)
    pltpu.sync_copy(neg_hbm.at[wid], nidx)

    copies = []
    for j in range(NCH):
        sl = pl.ds(j * CHUNK, CHUNK)
        copies.append(pltpu.async_copy(ue_hbm.at[uidx.at[j]], ue_rows.at[sl], sem))
        copies.append(pltpu.async_copy(ie_hbm.at[pidx.at[j]], pe_rows.at[sl], sem))
        copies.append(pltpu.async_copy(ie_hbm.at[nidx.at[j]], ne_rows.at[sl], sem))
        copies.append(pltpu.async_copy(ub_hbm.at[uidx.at[j]], ub_v.at[sl], sem))
        copies.append(pltpu.async_copy(ib_hbm.at[pidx.at[j]], pb_v.at[sl], sem))
        copies.append(pltpu.async_copy(ib_hbm.at[nidx.at[j]], nb_v.at[sl], sem))
    for c in copies:
        c.wait()

    lanes = lax.iota(jnp.int32, L)

    @pl.loop(0, GROUPS)
    def _group(g):
        pos = g * L + lanes
        accp = jnp.zeros((L,), jnp.float32)
        accn = jnp.zeros((L,), jnp.float32)
        for dd in range(D):
            col = jnp.full((L,), dd, jnp.int32)
            u = plsc.load_gather(ue_rows, [pos, col])
            p = plsc.load_gather(pe_rows, [pos, col])
            n = plsc.load_gather(ne_rows, [pos, col])
            accp = accp + u * p
            accn = accn + u * n
        sl = pl.ds(g * L, L)
        ub = ub_v[sl]
        outp_v[sl] = accp + ub + pb_v[sl]
        outn_v[sl] = accn + ub + nb_v[sl]

    pltpu.sync_copy(outp_v, outp_hbm.at[pl.ds(wid * BPW, BPW)])
    pltpu.sync_copy(outn_v, outn_hbm.at[pl.ds(wid * BPW, BPW)])


def kernel(users, pos_items, neg_items, user_embeddings, item_embeddings,
           user_biases, item_biases):
    users3 = users.astype(jnp.int32).reshape(NW, NCH, CHUNK)
    pos3 = pos_items.astype(jnp.int32).reshape(NW, NCH, CHUNK)
    neg3 = neg_items.astype(jnp.int32).reshape(NW, NCH, CHUNK)
    ub1 = user_biases.reshape(-1)
    ib1 = item_biases.reshape(-1)

    mesh = plsc.VectorSubcoreMesh(core_axis_name="core",
                                  subcore_axis_name="subcore",
                                  num_cores=NC, num_subcores=NS)
    f = pl.kernel(
        _sc_kernel,
        compiler_params=pltpu.CompilerParams(needs_layout_passes=False,
                                             use_tc_tiling_on_sc=False),
        out_type=(jax.ShapeDtypeStruct((B,), jnp.float32),
                  jax.ShapeDtypeStruct((B,), jnp.float32)),
        mesh=mesh,
        scratch_types=[
            pltpu.VMEM((NCH, CHUNK), jnp.int32),
            pltpu.VMEM((NCH, CHUNK), jnp.int32),
            pltpu.VMEM((NCH, CHUNK), jnp.int32),
            pltpu.VMEM((BPW, D), jnp.float32),
            pltpu.VMEM((BPW, D), jnp.float32),
            pltpu.VMEM((BPW, D), jnp.float32),
            pltpu.VMEM((BPW,), jnp.float32),
            pltpu.VMEM((BPW,), jnp.float32),
            pltpu.VMEM((BPW,), jnp.float32),
            pltpu.VMEM((BPW,), jnp.float32),
            pltpu.VMEM((BPW,), jnp.float32),
            pltpu.SemaphoreType.DMA,
        ],
    )
    pos_preds, neg_preds = f(users3, pos3, neg3, user_embeddings,
                             item_embeddings, ub1, ib1)
    return pos_preds, neg_preds
